# baseline (speedup 1.0000x reference)
import functools

import jax
import jax.numpy as jnp
from jax.experimental import pallas as pl
from jax.experimental.pallas import tpu as pltpu

B, S, H, T, NH, NL = 4, 512, 1024, 8, 16, 9
HF = 512
D = H + HF
EPS = 1e-12
TPB = 2
NTP = T // TPB
assert NTP == B


def _dot_t(a, b):
    return jax.lax.dot_general(a, b, (((1,), (1,)), ((), ())),
                               preferred_element_type=jnp.float32)


def _fused_kernel(tag_ref, wv_ref, bv_ref, opw_ref, opb_ref, w1_ref,
                  we_ref, st_ref, en_ref, b1_ref, w2_ref, b2_ref,
                  lng_ref, lnb_ref, wout_ref, bout_ref,
                  out_ref, attn_ref, g_ref, dr_ref, mom_ref):
    i = pl.program_id(0)

    @pl.when(i == 0)
    def _():
        vh = _dot_t(tag_ref[...], wv_ref[...]) + bv_ref[...]
        attn_ref[...] = _dot_t(vh, opw_ref[...]) + opb_ref[...]

    @pl.when(i < NTP)
    def _():
        for k in range(TPB):
            t = i * TPB + k
            at = attn_ref[pl.ds(t, 1), :]
            g_ref[pl.ds(t, 1), :] = _dot_t(at, w1_ref[:, k * H:(k + 1) * H])
        raw = we_ref[0]
        sw_a = wout_ref[:, :H] * lng_ref[:, :H]
        dr_ref[pl.ds(i, 1)] = _dot_t(raw, sw_a)[None]
        mom_ref[pl.ds(i, 1), :, 0:1] = jnp.sum(raw, -1,
                                               keepdims=True)[None]
        mom_ref[pl.ds(i, 1), :, 1:2] = jnp.sum(raw * raw, -1,
                                               keepdims=True)[None]

    @pl.when(i >= NTP)
    def _():
        b = i - NTP
        pos = jax.lax.broadcasted_iota(jnp.int32, (S, T), 0)
        starts = st_ref[0]
        ends = en_ref[0]
        mask = ((pos >= starts) & (pos < ends)).astype(jnp.bfloat16)
        hpre = jnp.dot(mask, g_ref[...].astype(jnp.bfloat16),
                       preferred_element_type=jnp.float32) + b1_ref[...]
        h = jnp.maximum(hpre, 0.0).astype(jnp.bfloat16)
        to = (_dot_t(h, w2_ref[...].astype(jnp.bfloat16))
              + b2_ref[...])
        dr = dr_ref[pl.ds(b, 1)][0]
        s1 = mom_ref[pl.ds(b, 1)][0, :, 0:1]
        s2 = mom_ref[pl.ds(b, 1)][0, :, 1:2]
        mu = (s1 + jnp.sum(to, -1, keepdims=True)) * (1.0 / D)
        ex2 = (s2 + jnp.sum(to * to, -1, keepdims=True)) * (1.0 / D)
        r = jax.lax.rsqrt(ex2 - mu * mu + EPS)
        sw = wout_ref[...] * lng_ref[...]
        dt = _dot_t(to.astype(jnp.bfloat16),
                    sw[:, H:].astype(jnp.bfloat16))
        csum = jnp.sum(sw, axis=1).reshape(1, NL)
        cvec = _dot_t(lnb_ref[...], wout_ref[...]) + bout_ref[...]
        out_ref[0] = r * (dr + dt) - (r * mu) * csum + cvec


@functools.partial(jax.jit, static_argnums=())
def kernel(word_embedding, spans, tag_embedding, in_proj_w, in_proj_b,
           out_proj_w, out_proj_b, W1, b1, W2, b2, ln_g, ln_b, Wout, bout):
    f32 = jnp.float32
    bv = in_proj_b[2 * H:].reshape(1, H)
    opb = out_proj_b.reshape(1, H)
    starts = spans[:, :, 0].astype(jnp.int32).reshape(B, 1, T)
    ends = spans[:, :, 1].astype(jnp.int32).reshape(B, 1, T)

    const = lambda i: (0, 0)
    wv_map = lambda i: (2, 0)
    bmap = lambda i: (jnp.maximum(i - NTP, 0), 0, 0)
    we_map = lambda i: (jnp.minimum(i, B - 1), 0, 0)

    out = pl.pallas_call(
        _fused_kernel,
        grid=(NTP + B,),
        in_specs=[
            pl.BlockSpec((T, H), const),
            pl.BlockSpec((H, H), wv_map),
            pl.BlockSpec((1, H), const),
            pl.BlockSpec((H, H), const),
            pl.BlockSpec((1, H), const),
            pl.BlockSpec((H, TPB * H), lambda i: (0, jnp.minimum(i, NTP - 1))),
            pl.BlockSpec((1, S, H), we_map),
            pl.BlockSpec((1, 1, T), bmap),
            pl.BlockSpec((1, 1, T), bmap),
            pl.BlockSpec((1, H), const),
            pl.BlockSpec((HF, H), const),
            pl.BlockSpec((1, HF), const),
            pl.BlockSpec((1, D), const),
            pl.BlockSpec((1, D), const),
            pl.BlockSpec((NL, D), const),
            pl.BlockSpec((1, NL), const),
        ],
        out_specs=pl.BlockSpec((1, S, NL), bmap),
        out_shape=jax.ShapeDtypeStruct((B, S, NL), f32),
        scratch_shapes=[pltpu.VMEM((T, H), f32), pltpu.VMEM((T, H), f32),
                        pltpu.VMEM((B, S, NL), f32),
                        pltpu.VMEM((B, S, 2), f32)],
    )(tag_embedding.astype(f32), in_proj_w, bv, out_proj_w, opb, W1,
      word_embedding, starts, ends, b1.reshape(1, H), W2,
      b2.reshape(1, HF), ln_g.reshape(1, D), ln_b.reshape(1, D),
      Wout, bout.reshape(1, NL))
    return out


# speedup vs baseline: 1.1289x; 1.1289x over previous
import functools

import jax
import jax.numpy as jnp
from jax.experimental import pallas as pl
from jax.experimental.pallas import tpu as pltpu

B, S, H, T, NH, NL = 4, 512, 1024, 8, 16, 9
HF = 512
D = H + HF
EPS = 1e-12
TPB = 2
NTP = T // TPB


def _dot_t(a, b):
    return jax.lax.dot_general(a, b, (((1,), (1,)), ((), ())),
                               preferred_element_type=jnp.float32)


def _fused_kernel(tag_ref, wv_ref, bv_ref, opw_ref, opb_ref, w1_ref,
                  we_ref, st_ref, en_ref, b1_ref, w2_ref, b2_ref,
                  lng_ref, lnb_ref, wout_ref, bout_ref,
                  out_ref, attn_ref, g_ref):
    i = pl.program_id(0)

    @pl.when(i == 0)
    def _():
        vh = _dot_t(tag_ref[...], wv_ref[...]) + bv_ref[...]
        attn_ref[...] = _dot_t(vh, opw_ref[...]) + opb_ref[...]

    @pl.when(i < NTP)
    def _():
        for k in range(TPB):
            t = i * TPB + k
            at = attn_ref[pl.ds(t, 1), :]
            g_ref[pl.ds(t, 1), :] = _dot_t(at, w1_ref[:, k * H:(k + 1) * H])

    @pl.when(i >= NTP)
    def _():
        raw = we_ref[0]
        pos = jax.lax.broadcasted_iota(jnp.int32, (S, T), 0)
        starts = st_ref[0]
        ends = en_ref[0]
        mask = ((pos >= starts) & (pos < ends)).astype(jnp.bfloat16)
        hpre = jnp.dot(mask, g_ref[...].astype(jnp.bfloat16),
                       preferred_element_type=jnp.float32) + b1_ref[...]
        h = jnp.maximum(hpre, 0.0).astype(jnp.bfloat16)
        to = (_dot_t(h, w2_ref[...].astype(jnp.bfloat16))
              + b2_ref[...])
        mu = (jnp.sum(raw, -1, keepdims=True)
              + jnp.sum(to, -1, keepdims=True)) * (1.0 / D)
        ex2 = (jnp.sum(raw * raw, -1, keepdims=True)
               + jnp.sum(to * to, -1, keepdims=True)) * (1.0 / D)
        r = jax.lax.rsqrt(ex2 - mu * mu + EPS)
        sw = wout_ref[...] * lng_ref[...]
        dr = _dot_t(raw, sw[:, :H])
        dt = _dot_t(to.astype(jnp.bfloat16),
                    sw[:, H:].astype(jnp.bfloat16))
        csum = jnp.sum(sw, axis=1).reshape(1, NL)
        cvec = _dot_t(lnb_ref[...], wout_ref[...]) + bout_ref[...]
        out_ref[0] = r * (dr + dt) - (r * mu) * csum + cvec


@functools.partial(jax.jit, static_argnums=())
def kernel(word_embedding, spans, tag_embedding, in_proj_w, in_proj_b,
           out_proj_w, out_proj_b, W1, b1, W2, b2, ln_g, ln_b, Wout, bout):
    f32 = jnp.float32
    bv = in_proj_b[2 * H:].reshape(1, H)
    opb = out_proj_b.reshape(1, H)
    starts = spans[:, :, 0].astype(jnp.int32).reshape(B, 1, T)
    ends = spans[:, :, 1].astype(jnp.int32).reshape(B, 1, T)

    const = lambda i: (0, 0)
    wv_map = lambda i: (2, 0)
    bmap = lambda i: (jnp.maximum(i - NTP, 0), 0, 0)

    out = pl.pallas_call(
        _fused_kernel,
        grid=(NTP + B,),
        in_specs=[
            pl.BlockSpec((T, H), const),
            pl.BlockSpec((H, H), wv_map),
            pl.BlockSpec((1, H), const),
            pl.BlockSpec((H, H), const),
            pl.BlockSpec((1, H), const),
            pl.BlockSpec((H, TPB * H), lambda i: (0, jnp.minimum(i, NTP - 1))),
            pl.BlockSpec((1, S, H), bmap),
            pl.BlockSpec((1, 1, T), bmap),
            pl.BlockSpec((1, 1, T), bmap),
            pl.BlockSpec((1, H), const),
            pl.BlockSpec((HF, H), const),
            pl.BlockSpec((1, HF), const),
            pl.BlockSpec((1, D), const),
            pl.BlockSpec((1, D), const),
            pl.BlockSpec((NL, D), const),
            pl.BlockSpec((1, NL), const),
        ],
        out_specs=pl.BlockSpec((1, S, NL), bmap),
        out_shape=jax.ShapeDtypeStruct((B, S, NL), f32),
        scratch_shapes=[pltpu.VMEM((T, H), f32), pltpu.VMEM((T, H), f32)],
    )(tag_embedding.astype(f32), in_proj_w, bv, out_proj_w, opb, W1,
      word_embedding, starts, ends, b1.reshape(1, H), W2,
      b2.reshape(1, HF), ln_g.reshape(1, D), ln_b.reshape(1, D),
      Wout, bout.reshape(1, NL))
    return out
